# baseline (speedup 1.0000x reference)
import jax
import jax.numpy as jnp
from jax import lax
from jax.experimental import pallas as pl
from jax.experimental.pallas import tpu as pltpu
from jax.experimental.pallas import tpu_sc as plsc

N = 10000
D = 128
E = 320000
NC = 2
NS = 16
NW = NC * NS
EDGES_PER_TILE = E // NW
CHUNK = 128
NCH = 80
PAD = NCH * CHUNK - EDGES_PER_TILE
GROUPS = NCH // 4
ROWS_PER_TILE = 624
ROWS_TAIL = N - NS * ROWS_PER_TILE
SUP_ROWS = N + 8


def _mm_body(x_ref, w_ref, o_ref):
    o_ref[pl.ds(0, N), :] = jnp.dot(x_ref[...], w_ref[...],
                                    preferred_element_type=jnp.float32)
    o_ref[pl.ds(N, SUP_ROWS - N), :] = jnp.zeros((SUP_ROWS - N, D),
                                                 jnp.float32)


def _matmul(x, w):
    return pl.pallas_call(
        _mm_body,
        out_shape=jax.ShapeDtypeStruct((SUP_ROWS, w.shape[1]), jnp.float32),
    )(x, w)


def _merge_mm_body(p_ref, b_ref, w_ref, o_ref):
    h = p_ref[0] + p_ref[1] + b_ref[...]
    h = jnp.where(h >= 0, h, 0.25 * h)
    o_ref[pl.ds(0, N), :] = jnp.dot(h, w_ref[...],
                                    preferred_element_type=jnp.float32)
    o_ref[pl.ds(N, SUP_ROWS - N), :] = jnp.zeros((SUP_ROWS - N, D),
                                                 jnp.float32)


def _merge_matmul(partials, b, w):
    return pl.pallas_call(
        _merge_mm_body,
        out_shape=jax.ShapeDtypeStruct((SUP_ROWS, w.shape[1]), jnp.float32),
    )(partials, b, w)


def _merge_act_body(p_ref, b_ref, o_ref):
    h = p_ref[0] + p_ref[1] + b_ref[...]
    o_ref[...] = jnp.where(h >= 0, h, 0.25 * h)


def _merge_act(partials, b):
    return pl.pallas_call(
        _merge_act_body,
        out_shape=jax.ShapeDtypeStruct((N, D), jnp.float32),
    )(partials, b)


def _sc_scatter_body(sup_hbm, row_hbm, col_hbm, zero_hbm, out_hbm, *refs):
    colv = refs[0:4]
    rowv = refs[4:8]
    gat = refs[8:10]
    ics = refs[10:14]
    irs = refs[14:18]
    gs = refs[18:20]
    ss = refs[20:24]
    acc = refs[24]
    cid = lax.axis_index("c")
    sid = lax.axis_index("s")
    wid = cid * NS + sid
    rbase = sid * ROWS_PER_TILE
    ebase = wid * (NCH * CHUNK)

    pltpu.sync_copy(zero_hbm.at[pl.ds(rbase, ROWS_PER_TILE)],
                    acc.at[pl.ds(rbase, ROWS_PER_TILE)])

    @pl.when(sid == NS - 1)
    def _():
        pltpu.sync_copy(zero_hbm.at[pl.ds(NS * ROWS_PER_TILE, ROWS_TAIL)],
                        acc.at[pl.ds(NS * ROWS_PER_TILE, ROWS_TAIL)])

    plsc.subcore_barrier()

    def idx_issue(jj, k):
        off = ebase + jnp.minimum(jj, NCH - 1) * CHUNK
        pltpu.async_copy(col_hbm.at[pl.ds(off, CHUNK)], colv[k], ics[k])
        pltpu.async_copy(row_hbm.at[pl.ds(off, CHUNK)], rowv[k], irs[k])

    def drain_ic(k):
        pltpu.make_async_copy(col_hbm.at[pl.ds(0, CHUNK)], colv[k],
                              ics[k]).wait()

    def drain_ir(k):
        pltpu.make_async_copy(row_hbm.at[pl.ds(0, CHUNK)], rowv[k],
                              irs[k]).wait()

    def gather_issue(kc, kg):
        pltpu.async_copy(sup_hbm.at[colv[kc]], gat[kg], gs[kg])

    def drain_g(kc, kg):
        pltpu.make_async_copy(sup_hbm.at[colv[kc]], gat[kg], gs[kg]).wait()

    def scatter_issue(kg, kr):
        pltpu.async_copy(gat[kg], acc.at[rowv[kr]], ss[kr], add=True)

    def drain_s(kg, kr):
        pltpu.make_async_copy(gat[kg], acc.at[rowv[kr]], ss[kr]).wait()

    def chunk_step(jj, k, first):
        cg = k % 2
        og = 1 - cg
        drain_g(k, cg)
        if not (first and k == 0):
            drain_s(og, (k - 1) % 4)
        drain_ic((k + 1) % 4)
        gather_issue((k + 1) % 4, og)
        drain_ir(k)
        scatter_issue(cg, k)
        idx_issue(jj + 3, (k + 3) % 4)

    idx_issue(0, 0)
    idx_issue(1, 1)
    idx_issue(2, 2)
    drain_ic(0)
    gather_issue(0, 0)
    for k in range(4):
        chunk_step(k, k, first=True)

    @pl.loop(1, GROUPS)
    def _(m):
        base = 4 * m
        for k in range(4):
            chunk_step(base + k, k, first=False)

    drain_g(0, 0)
    drain_s(1, 3)
    drain_ir(0)
    drain_ic(1)
    drain_ir(1)
    drain_ic(2)
    drain_ir(2)

    plsc.subcore_barrier()
    pltpu.sync_copy(acc.at[pl.ds(rbase, ROWS_PER_TILE)],
                    out_hbm.at[cid, pl.ds(rbase, ROWS_PER_TILE)])

    @pl.when(sid == NS - 1)
    def _():
        pltpu.sync_copy(acc.at[pl.ds(NS * ROWS_PER_TILE, ROWS_TAIL)],
                        out_hbm.at[cid, pl.ds(NS * ROWS_PER_TILE, ROWS_TAIL)])


def _sc_scatter_add(support, rowf, colf, zeros):
    mesh = plsc.VectorSubcoreMesh(core_axis_name="c", subcore_axis_name="s")
    k = pl.kernel(
        _sc_scatter_body,
        out_type=jax.ShapeDtypeStruct((NC, N, D), jnp.float32),
        mesh=mesh,
        scratch_types=(
            [pltpu.VMEM((CHUNK,), jnp.int32)] * 8
            + [pltpu.VMEM((CHUNK, D), jnp.float32)] * 2
            + [pltpu.SemaphoreType.DMA] * 14
            + [pltpu.VMEM_SHARED((N, D), jnp.float32)]
        ),
    )
    return k(support, rowf, colf, zeros)


def kernel(x, edge_index, W1, b1, W2, b2):
    ei = edge_index.astype(jnp.int32)
    rowf = jnp.pad(ei[0].reshape(NW, EDGES_PER_TILE), ((0, 0), (0, PAD)),
                   constant_values=0).reshape(NW * NCH * CHUNK)
    colf = jnp.pad(ei[1].reshape(NW, EDGES_PER_TILE), ((0, 0), (0, PAD)),
                   constant_values=N).reshape(NW * NCH * CHUNK)
    zeros = jnp.zeros((N, D), jnp.float32)
    b1r = jnp.reshape(b1, (1, D))
    b2r = jnp.reshape(b2, (1, D))

    support1 = _matmul(x, W1)
    part1 = _sc_scatter_add(support1, rowf, colf, zeros)
    support2 = _merge_matmul(part1, b1r, W2)
    part2 = _sc_scatter_add(support2, rowf, colf, zeros)
    return _merge_act(part2, b2r)


# speedup vs baseline: 1.2251x; 1.2251x over previous
import jax
import jax.numpy as jnp
from jax import lax
from jax.experimental import pallas as pl
from jax.experimental.pallas import tpu as pltpu
from jax.experimental.pallas import tpu_sc as plsc

N = 10000
D = 128
E = 320000
NC = 2
NS = 16
NW = NC * NS
EDGES_PER_TILE = E // NW
CHUNK = 128
NCH = 80
NCH2 = NCH // 2
PAD = NCH * CHUNK - EDGES_PER_TILE
ROWS_PER_TILE = 624
ROWS_TAIL = N - NS * ROWS_PER_TILE
SUP_ROWS = N + 8


def _mm_body(x_ref, w_ref, o_ref):
    o_ref[pl.ds(0, N), :] = jnp.dot(x_ref[...], w_ref[...],
                                    preferred_element_type=jnp.float32)
    o_ref[pl.ds(N, SUP_ROWS - N), :] = jnp.zeros((SUP_ROWS - N, D),
                                                 jnp.float32)


def _matmul(x, w):
    return pl.pallas_call(
        _mm_body,
        out_shape=jax.ShapeDtypeStruct((SUP_ROWS, w.shape[1]), jnp.float32),
    )(x, w)


def _merge_mm_body(p_ref, b_ref, w_ref, o_ref):
    h = p_ref[0] + p_ref[1] + b_ref[...]
    h = jnp.where(h >= 0, h, 0.25 * h)
    o_ref[pl.ds(0, N), :] = jnp.dot(h, w_ref[...],
                                    preferred_element_type=jnp.float32)
    o_ref[pl.ds(N, SUP_ROWS - N), :] = jnp.zeros((SUP_ROWS - N, D),
                                                 jnp.float32)


def _merge_matmul(partials, b, w):
    return pl.pallas_call(
        _merge_mm_body,
        out_shape=jax.ShapeDtypeStruct((SUP_ROWS, w.shape[1]), jnp.float32),
    )(partials, b, w)


def _merge_act_body(p_ref, b_ref, o_ref):
    h = p_ref[0] + p_ref[1] + b_ref[...]
    o_ref[...] = jnp.where(h >= 0, h, 0.25 * h)


def _merge_act(partials, b):
    return pl.pallas_call(
        _merge_act_body,
        out_shape=jax.ShapeDtypeStruct((N, D), jnp.float32),
    )(partials, b)


def _sc_scatter_body(sup_hbm, rowp_hbm, colp_hbm, zero_hbm, out_hbm,
                     colv, rowv, gat, acc):
    cid = lax.axis_index("c")
    sid = lax.axis_index("s")
    wid = cid * NS + sid
    rbase = sid * ROWS_PER_TILE

    pltpu.sync_copy(zero_hbm.at[pl.ds(rbase, ROWS_PER_TILE)],
                    acc.at[pl.ds(rbase, ROWS_PER_TILE)])

    @pl.when(sid == NS - 1)
    def _():
        pltpu.sync_copy(zero_hbm.at[pl.ds(NS * ROWS_PER_TILE, ROWS_TAIL)],
                        acc.at[pl.ds(NS * ROWS_PER_TILE, ROWS_TAIL)])

    plsc.subcore_barrier()

    for h in range(2):
        pltpu.sync_copy(colp_hbm.at[wid, pl.ds(h * NCH2, NCH2)], colv)
        pltpu.sync_copy(rowp_hbm.at[wid, pl.ds(h * NCH2, NCH2)], rowv)

        @pl.loop(0, NCH2)
        def _(j):
            pltpu.sync_copy(sup_hbm.at[colv.at[j]], gat)
            pltpu.sync_copy(gat, acc.at[rowv.at[j]], add=True)

    plsc.subcore_barrier()
    pltpu.sync_copy(acc.at[pl.ds(rbase, ROWS_PER_TILE)],
                    out_hbm.at[cid, pl.ds(rbase, ROWS_PER_TILE)])

    @pl.when(sid == NS - 1)
    def _():
        pltpu.sync_copy(acc.at[pl.ds(NS * ROWS_PER_TILE, ROWS_TAIL)],
                        out_hbm.at[cid, pl.ds(NS * ROWS_PER_TILE, ROWS_TAIL)])


def _sc_scatter_add(support, rowp, colp, zeros):
    mesh = plsc.VectorSubcoreMesh(core_axis_name="c", subcore_axis_name="s")
    k = pl.kernel(
        _sc_scatter_body,
        out_type=jax.ShapeDtypeStruct((NC, N, D), jnp.float32),
        mesh=mesh,
        scratch_types=[
            pltpu.VMEM((NCH2, CHUNK), jnp.int32),
            pltpu.VMEM((NCH2, CHUNK), jnp.int32),
            pltpu.VMEM((CHUNK, D), jnp.float32),
            pltpu.VMEM_SHARED((N, D), jnp.float32),
        ],
    )
    return k(support, rowp, colp, zeros)


def kernel(x, edge_index, W1, b1, W2, b2):
    ei = edge_index.astype(jnp.int32)
    rowp = jnp.pad(ei[0].reshape(NW, EDGES_PER_TILE), ((0, 0), (0, PAD)),
                   constant_values=0).reshape(NW, NCH, CHUNK)
    colp = jnp.pad(ei[1].reshape(NW, EDGES_PER_TILE), ((0, 0), (0, PAD)),
                   constant_values=N).reshape(NW, NCH, CHUNK)
    zeros = jnp.zeros((N, D), jnp.float32)
    b1r = jnp.reshape(b1, (1, D))
    b2r = jnp.reshape(b2, (1, D))

    support1 = _matmul(x, W1)
    part1 = _sc_scatter_add(support1, rowp, colp, zeros)
    support2 = _merge_matmul(part1, b1r, W2)
    part2 = _sc_scatter_add(support2, rowp, colp, zeros)
    return _merge_act(part2, b2r)
